# baseline (speedup 1.0000x reference)
import jax
import jax.numpy as jnp
from jax import lax
from jax.experimental import pallas as pl
from jax.experimental.pallas import tpu as pltpu
from jax.experimental.pallas import tpu_sc as plsc

HE = 10000
D = 128
NP = 10240

_NC = 2
_NS = 16
_TILES = _NC * _NS
_C = 80
_G = _C // 16
_NB = 2


def _bcast_lane(v16, i):
    return jnp.take_along_axis(v16, jnp.full((16,), i, jnp.int32), axis=0)


def _zero_rows(ref, nrows):
    z = jnp.zeros((16,), jnp.float32)

    def body(r, carry):
        for b in range(D // 16):
            ref[r, pl.ds(b * 16, 16)] = z
        return carry

    lax.fori_loop(0, nrows, body, 0)


def _scale_rows(src, dst, scal16, g):
    for i in range(16):
        bv = _bcast_lane(scal16, i)
        r = g * 16 + i
        for b in range(D // 16):
            dst[r, pl.ds(b * 16, 16)] = src[r, pl.ds(b * 16, 16)] * bv


def _zero_acc_slice(acc, rows_buf, s):
    _zero_rows(rows_buf, _C)
    rpt = NP // _NS
    for j in range(rpt // _C):
        r0 = pl.multiple_of(s * rpt + j * _C, 8)
        pltpu.sync_copy(rows_buf, acc.at[pl.ds(r0, _C)])


def _stage_writeback(acc, parts_hbm, rows_buf, c, s):
    rpt = NP // _NS
    for j in range(rpt // _C):
        r0 = pl.multiple_of(s * rpt + j * _C, 8)
        pltpu.sync_copy(acc.at[pl.ds(r0, _C)], rows_buf)
        pltpu.sync_copy(rows_buf, parts_hbm.at[c, pl.ds(r0, _C)])


def _edge_accum_body(nid_hbm, hid_hbm, val_hbm, x_hbm, eparts_hbm,
                     e_acc, hid_v, nidb0, nidb1, lidb0, lidb1, valb0, valb1,
                     rin0, rin1, rout,
                     semg0, semg1, semv0, semv1, semn0, semn1):
    c = lax.axis_index("c")
    s = lax.axis_index("s")
    wid = c * _NS + s
    nch = hid_v.shape[0]
    per_tile = nch * _C
    base = wid * per_tile
    rin = (rin0, rin1)
    nidb = (nidb0, nidb1)
    lidb = (lidb0, lidb1)
    valb = (valb0, valb1)
    semg = (semg0, semg1)
    semv = (semv0, semv1)
    semn = (semn0, semn1)

    pltpu.sync_copy(hid_hbm.at[wid], hid_v)

    iota16 = lax.iota(jnp.int32, 16)
    prev_idx = jnp.maximum(iota16 - 1, 0)

    def start_idx(k, p):
        off = pl.multiple_of(base + k * _C, 8)
        pltpu.async_copy(nid_hbm.at[pl.ds(off, _C)], nidb[p], semn[p])

    def start_val(k, p):
        off = pl.multiple_of(base + k * _C, 8)
        pltpu.async_copy(val_hbm.at[pl.ds(off, _C)], valb[p], semv[p])

    def start_gather(p):
        pltpu.make_async_copy(nid_hbm.at[pl.ds(0, _C)], nidb[p], semn[p]).wait()
        carry_last = jnp.full((16,), -1, jnp.int32)
        carry_fo = jnp.zeros((16,), jnp.int32)
        for g in range(_G):
            i16 = nidb[p][pl.ds(g * 16, 16)]
            prev16 = jnp.take_along_axis(i16, prev_idx, axis=0)
            prev16 = jnp.where(iota16 == 0, carry_last, prev16)
            first16 = i16 != prev16
            pos16 = jnp.where(first16, iota16 + g * 16, 0)
            fo16 = jnp.maximum(plsc.cummax(pos16), carry_fo)
            nidb[p][pl.ds(g * 16, 16)] = jnp.where(first16, i16, iota16 + g * 16)
            lidb[p][pl.ds(g * 16, 16)] = fo16
            carry_last = _bcast_lane(i16, 15)
            carry_fo = _bcast_lane(fo16, 15)
        pltpu.async_copy(x_hbm.at[nidb[p]], rin[p], semg[p])

    for p in range(_NB):
        start_idx(p, p)
        start_val(p, p)
    for p in range(_NB):
        start_gather(p)

    _zero_acc_slice(e_acc, rout, s)
    plsc.subcore_barrier()

    def expand_scale(p, g):
        v16 = valb[p][pl.ds(g * 16, 16)]
        lid16 = lidb[p][pl.ds(g * 16, 16)]
        for i in range(16):
            bv = _bcast_lane(v16, i)
            row16 = _bcast_lane(lid16, i)
            r = g * 16 + i
            for b in range(D // 16):
                blk = plsc.load_gather(rin[p], [row16, iota16 + b * 16])
                rout[r, pl.ds(b * 16, 16)] = blk * bv

    def visit(k, p, guarded_prefetch):
        pltpu.make_async_copy(x_hbm.at[nidb[p]], rin[p], semg[p]).wait()
        pltpu.make_async_copy(val_hbm.at[pl.ds(0, _C)], valb[p], semv[p]).wait()

        def do_prefetch():
            start_idx(k + _NB, p)

        if guarded_prefetch:
            pl.when(k + _NB < nch)(do_prefetch)

        def grp(g, carry):
            expand_scale(p, g)
            return carry

        lax.fori_loop(0, _G, grp, 0)

        def do_gather():
            start_gather(p)
            start_val(k + _NB, p)

        if guarded_prefetch:
            pl.when(k + _NB < nch)(do_gather)
        pltpu.sync_copy(rout, e_acc.at[hid_v.at[k]], add=True)

    def steady(kk, carry):
        visit(kk * _NB, 0, True)
        visit(kk * _NB + 1, 1, True)
        return carry

    lax.fori_loop(0, (nch - 1) // _NB, steady, 0)
    visit(nch - 1, 0, False)

    plsc.subcore_barrier()
    _stage_writeback(e_acc, eparts_hbm, rin0, c, s)


def _attn_accum_body(nid_hbm, hid_hbm, w_hbm, et_hbm,
                     yparts_hbm, dparts_hbm,
                     y_acc, d_acc, nid_v, hid_v,
                     wb0, wb1, rin0, rin1, dbuf,
                     semg0, semg1, semw0, semw1):
    c = lax.axis_index("c")
    s = lax.axis_index("s")
    wid = c * _NS + s
    nch = nid_v.shape[0]
    per_tile = nch * _C
    base = wid * per_tile
    rin = (rin0, rin1)
    wb = (wb0, wb1)
    semg = (semg0, semg1)
    semw = (semw0, semw1)

    b0 = pl.multiple_of(base, 8)
    pltpu.sync_copy(nid_hbm.at[wid], nid_v)
    pltpu.sync_copy(hid_hbm.at[pl.ds(b0, per_tile)], hid_v)

    _zero_acc_slice(y_acc, rin0, s)
    z16 = jnp.zeros((16,), jnp.float32)
    rpt = NP // _NS

    def zd(i, carry):
        dbuf[pl.ds(i * 16, 16)] = z16
        return carry

    lax.fori_loop(0, 8, zd, 0)
    d0 = pl.multiple_of(s * rpt, 8)
    for j in range(rpt // 128):
        pltpu.sync_copy(dbuf, d_acc.at[pl.ds(pl.multiple_of(d0 + j * 128, 8), 128)])

    def start_fetch(k, p):
        hslice = hid_v.at[pl.ds(k * _C, _C)]
        pltpu.async_copy(et_hbm.at[hslice], rin[p], semg[p])
        pltpu.async_copy(w_hbm.at[hslice], wb[p], semw[p])

    def wait_fetch(k, p):
        hslice = hid_v.at[pl.ds(k * _C, _C)]
        pltpu.make_async_copy(et_hbm.at[hslice], rin[p], semg[p]).wait()
        pltpu.make_async_copy(w_hbm.at[hslice], wb[p], semw[p]).wait()

    for p in range(_NB):
        start_fetch(p, p)
    plsc.subcore_barrier()

    def visit(k, p, prefetch):
        wait_fetch(k, p)
        pltpu.sync_copy(wb[p], d_acc.at[nid_v.at[k]], add=True)
        pltpu.sync_copy(rin[p], y_acc.at[nid_v.at[k]], add=True)
        if prefetch:
            start_fetch(k + _NB, p)

    def steady(kk, carry):
        visit(kk * _NB, 0, True)
        visit(kk * _NB + 1, 1, True)
        return carry

    lax.fori_loop(0, (nch - 3) // _NB, steady, 0)
    visit(nch - 3, 0, True)
    visit(nch - 2, 1, False)
    visit(nch - 1, 0, False)

    plsc.subcore_barrier()
    _stage_writeback(y_acc, yparts_hbm, rin0, c, s)
    for j in range(rpt // 128):
        dj = pl.multiple_of(d0 + j * 128, 8)
        pltpu.sync_copy(d_acc.at[pl.ds(dj, 128)], dbuf)
        pltpu.sync_copy(dbuf, dparts_hbm.at[c, pl.ds(dj, 128)])


def _project_body(ep_ref, a2_ref, et_ref, w_ref):
    ep = ep_ref[0] + ep_ref[1]
    t = jnp.sum(ep * a2_ref[...], axis=1, keepdims=True)
    w = jnp.exp(t)
    w_ref[...] = w
    et_ref[...] = ep * w


def _finalize_body(yp_ref, dp_ref, out_ref):
    y = yp_ref[0] + yp_ref[1]
    dsum = dp_ref[0] + dp_ref[1]
    out_ref[...] = y / (dsum + 1e-16)


def kernel(H_indices, H_values, X, a):
    n_nodes, d = X.shape
    nnz = H_values.shape[0]
    per_tile = nnz // _TILES
    nch = per_tile // _C
    nid_flat = H_indices[0].astype(jnp.int32)
    hid_flat = H_indices[1].astype(jnp.int32)
    nid3 = nid_flat.reshape(_TILES, nch, _C)
    hid3 = hid_flat.reshape(_TILES, nch, _C)
    vals = H_values.astype(jnp.float32)

    mesh = plsc.VectorSubcoreMesh(core_axis_name="c", subcore_axis_name="s")
    sc_params = pltpu.CompilerParams(needs_layout_passes=False)

    edge_accum = pl.kernel(
        _edge_accum_body,
        out_type=jax.ShapeDtypeStruct((_NC, NP, D), jnp.float32),
        mesh=mesh,
        compiler_params=sc_params,
        scratch_types=[
            pltpu.VMEM_SHARED((NP, D), jnp.float32),
            pltpu.VMEM((nch, _C), jnp.int32),
            pltpu.VMEM((_C,), jnp.int32),
            pltpu.VMEM((_C,), jnp.int32),
            pltpu.VMEM((_C,), jnp.int32),
            pltpu.VMEM((_C,), jnp.int32),
            pltpu.VMEM((_C,), jnp.float32),
            pltpu.VMEM((_C,), jnp.float32),
            pltpu.VMEM((_C, D), jnp.float32),
            pltpu.VMEM((_C, D), jnp.float32),
            pltpu.VMEM((_C, D), jnp.float32),
            pltpu.SemaphoreType.DMA,
            pltpu.SemaphoreType.DMA,
            pltpu.SemaphoreType.DMA,
            pltpu.SemaphoreType.DMA,
            pltpu.SemaphoreType.DMA,
            pltpu.SemaphoreType.DMA,
        ],
    )
    e_parts = edge_accum(nid_flat, hid3, vals, X)

    r_blk = 1024
    et_full, w2 = pl.pallas_call(
        _project_body,
        grid=(NP // r_blk,),
        in_specs=[
            pl.BlockSpec((_NC, r_blk, D), lambda i: (0, i, 0)),
            pl.BlockSpec((1, D), lambda i: (0, 0)),
        ],
        out_specs=[
            pl.BlockSpec((r_blk, D), lambda i: (i, 0)),
            pl.BlockSpec((r_blk, 1), lambda i: (i, 0)),
        ],
        out_shape=[
            jax.ShapeDtypeStruct((NP, D), jnp.float32),
            jax.ShapeDtypeStruct((NP, 1), jnp.float32),
        ],
    )(e_parts, a[d:].reshape(1, d))
    w_tab = w2.reshape(-1)

    attn_accum = pl.kernel(
        _attn_accum_body,
        out_type=[
            jax.ShapeDtypeStruct((_NC, NP, D), jnp.float32),
            jax.ShapeDtypeStruct((_NC, NP), jnp.float32),
        ],
        mesh=mesh,
        compiler_params=sc_params,
        scratch_types=[
            pltpu.VMEM_SHARED((NP, D), jnp.float32),
            pltpu.VMEM_SHARED((NP,), jnp.float32),
            pltpu.VMEM((nch, _C), jnp.int32),
            pltpu.VMEM((per_tile,), jnp.int32),
            pltpu.VMEM((_C,), jnp.float32),
            pltpu.VMEM((_C,), jnp.float32),
            pltpu.VMEM((_C, D), jnp.float32),
            pltpu.VMEM((_C, D), jnp.float32),
            pltpu.VMEM((128,), jnp.float32),
            pltpu.SemaphoreType.DMA,
            pltpu.SemaphoreType.DMA,
            pltpu.SemaphoreType.DMA,
            pltpu.SemaphoreType.DMA,
        ],
    )
    y_parts, d_parts = attn_accum(nid3, hid_flat, w_tab, et_full)

    dp3 = d_parts.reshape(_NC, NP, 1)
    out_pad = pl.pallas_call(
        _finalize_body,
        grid=(NP // r_blk,),
        in_specs=[
            pl.BlockSpec((_NC, r_blk, D), lambda i: (0, i, 0)),
            pl.BlockSpec((_NC, r_blk, 1), lambda i: (0, i, 0)),
        ],
        out_specs=pl.BlockSpec((r_blk, D), lambda i: (i, 0)),
        out_shape=jax.ShapeDtypeStruct((NP, D), jnp.float32),
    )(y_parts, dp3)
    return out_pad[:n_nodes]


# speedup vs baseline: 1.2095x; 1.2095x over previous
import jax
import jax.numpy as jnp
from jax import lax
from jax.experimental import pallas as pl
from jax.experimental.pallas import tpu as pltpu
from jax.experimental.pallas import tpu_sc as plsc

HE = 10000
D = 128
NP = 10240

_NC = 2
_NS = 16
_TILES = _NC * _NS
_C = 80
_G = _C // 16
_NB = 2


def _bcast_lane(v16, i):
    return jnp.take_along_axis(v16, jnp.full((16,), i, jnp.int32), axis=0)


def _zero_rows(ref, nrows):
    z = jnp.zeros((16,), jnp.float32)

    def body(r, carry):
        for b in range(D // 16):
            ref[r, pl.ds(b * 16, 16)] = z
        return carry

    lax.fori_loop(0, nrows, body, 0)


def _scale_rows(src, dst, scal16, g):
    for i in range(16):
        bv = _bcast_lane(scal16, i)
        r = g * 16 + i
        for b in range(D // 16):
            dst[r, pl.ds(b * 16, 16)] = src[r, pl.ds(b * 16, 16)] * bv


def _zero_acc_slice(acc, rows_buf, s):
    _zero_rows(rows_buf, _C)
    rpt = NP // _NS
    for j in range(rpt // _C):
        r0 = pl.multiple_of(s * rpt + j * _C, 8)
        pltpu.sync_copy(rows_buf, acc.at[pl.ds(r0, _C)])


def _stage_writeback(acc, parts_hbm, rows_buf, c, s):
    rpt = NP // _NS
    for j in range(rpt // _C):
        r0 = pl.multiple_of(s * rpt + j * _C, 8)
        pltpu.sync_copy(acc.at[pl.ds(r0, _C)], rows_buf)
        pltpu.sync_copy(rows_buf, parts_hbm.at[c, pl.ds(r0, _C)])


def _edge_accum_body(nid_hbm, hid_hbm, val_hbm, x_hbm, eparts_hbm,
                     e_acc, hid_v, nidb0, nidb1, valb0, valb1,
                     rin0, rin1, rout,
                     semg0, semg1, semv0, semv1, semn0, semn1):
    c = lax.axis_index("c")
    s = lax.axis_index("s")
    wid = c * _NS + s
    nch = hid_v.shape[0]
    per_tile = nch * _C
    base = wid * per_tile
    rin = (rin0, rin1)
    nidb = (nidb0, nidb1)
    valb = (valb0, valb1)
    semg = (semg0, semg1)
    semv = (semv0, semv1)
    semn = (semn0, semn1)

    pltpu.sync_copy(hid_hbm.at[wid], hid_v)

    def start_idx(k, p):
        off = pl.multiple_of(base + k * _C, 8)
        pltpu.async_copy(nid_hbm.at[pl.ds(off, _C)], nidb[p], semn[p])

    def start_val(k, p):
        off = pl.multiple_of(base + k * _C, 8)
        pltpu.async_copy(val_hbm.at[pl.ds(off, _C)], valb[p], semv[p])

    def start_gather(p):
        pltpu.make_async_copy(nid_hbm.at[pl.ds(0, _C)], nidb[p], semn[p]).wait()
        pltpu.async_copy(x_hbm.at[nidb[p]], rin[p], semg[p])

    for p in range(_NB):
        start_idx(p, p)
        start_val(p, p)
    for p in range(_NB):
        start_gather(p)

    _zero_acc_slice(e_acc, rout, s)
    plsc.subcore_barrier()

    def expand_scale(p, g):
        v16 = valb[p][pl.ds(g * 16, 16)]
        _scale_rows(rin[p], rout, v16, g)

    def visit(k, p, prefetch, unroll):
        pltpu.make_async_copy(x_hbm.at[nidb[p]], rin[p], semg[p]).wait()
        pltpu.make_async_copy(val_hbm.at[pl.ds(0, _C)], valb[p], semv[p]).wait()
        if prefetch:
            start_idx(k + _NB, p)
        if unroll:
            for g in range(_G):
                expand_scale(p, g)
        else:
            def grp(g, carry):
                expand_scale(p, g)
                return carry

            lax.fori_loop(0, _G, grp, 0)
        if prefetch:
            start_gather(p)
            start_val(k + _NB, p)
        pltpu.sync_copy(rout, e_acc.at[hid_v.at[k]], add=True)

    def steady(kk, carry):
        visit(kk * _NB, 0, True, True)
        visit(kk * _NB + 1, 1, True, True)
        return carry

    lax.fori_loop(0, (nch - 3) // _NB, steady, 0)
    visit(nch - 3, 0, True, True)
    visit(nch - 2, 1, False, True)
    visit(nch - 1, 0, False, True)

    plsc.subcore_barrier()
    _stage_writeback(e_acc, eparts_hbm, rin0, c, s)


def _attn_accum_body(nid_hbm, hid_hbm, w_hbm, et_hbm,
                     yparts_hbm, dparts_hbm,
                     y_acc, d_acc, nid_v, hid_v,
                     wb0, wb1, rin0, rin1, dbuf,
                     semg0, semg1, semw0, semw1):
    c = lax.axis_index("c")
    s = lax.axis_index("s")
    wid = c * _NS + s
    nch = nid_v.shape[0]
    per_tile = nch * _C
    base = wid * per_tile
    rin = (rin0, rin1)
    wb = (wb0, wb1)
    semg = (semg0, semg1)
    semw = (semw0, semw1)

    b0 = pl.multiple_of(base, 8)
    pltpu.sync_copy(nid_hbm.at[wid], nid_v)
    pltpu.sync_copy(hid_hbm.at[pl.ds(b0, per_tile)], hid_v)

    _zero_acc_slice(y_acc, rin0, s)
    z16 = jnp.zeros((16,), jnp.float32)
    rpt = NP // _NS

    def zd(i, carry):
        dbuf[pl.ds(i * 16, 16)] = z16
        return carry

    lax.fori_loop(0, 8, zd, 0)
    d0 = pl.multiple_of(s * rpt, 8)
    for j in range(rpt // 128):
        pltpu.sync_copy(dbuf, d_acc.at[pl.ds(pl.multiple_of(d0 + j * 128, 8), 128)])

    def start_fetch(k, p):
        hslice = hid_v.at[pl.ds(k * _C, _C)]
        pltpu.async_copy(et_hbm.at[hslice], rin[p], semg[p])
        pltpu.async_copy(w_hbm.at[hslice], wb[p], semw[p])

    def wait_fetch(k, p):
        hslice = hid_v.at[pl.ds(k * _C, _C)]
        pltpu.make_async_copy(et_hbm.at[hslice], rin[p], semg[p]).wait()
        pltpu.make_async_copy(w_hbm.at[hslice], wb[p], semw[p]).wait()

    for p in range(_NB):
        start_fetch(p, p)
    plsc.subcore_barrier()

    def visit(k, p, prefetch):
        wait_fetch(k, p)
        pltpu.sync_copy(wb[p], d_acc.at[nid_v.at[k]], add=True)
        pltpu.sync_copy(rin[p], y_acc.at[nid_v.at[k]], add=True)
        if prefetch:
            start_fetch(k + _NB, p)

    def steady(kk, carry):
        visit(kk * _NB, 0, True)
        visit(kk * _NB + 1, 1, True)
        return carry

    lax.fori_loop(0, (nch - 3) // _NB, steady, 0)
    visit(nch - 3, 0, True)
    visit(nch - 2, 1, False)
    visit(nch - 1, 0, False)

    plsc.subcore_barrier()
    _stage_writeback(y_acc, yparts_hbm, rin0, c, s)
    for j in range(rpt // 128):
        dj = pl.multiple_of(d0 + j * 128, 8)
        pltpu.sync_copy(d_acc.at[pl.ds(dj, 128)], dbuf)
        pltpu.sync_copy(dbuf, dparts_hbm.at[c, pl.ds(dj, 128)])


def _project_body(ep_ref, a2_ref, et_ref, w_ref):
    ep = ep_ref[0] + ep_ref[1]
    t = jnp.sum(ep * a2_ref[...], axis=1, keepdims=True)
    w = jnp.exp(t)
    w_ref[...] = w
    et_ref[...] = ep * w


def _finalize_body(yp_ref, dp_ref, out_ref):
    y = yp_ref[0] + yp_ref[1]
    dsum = dp_ref[0] + dp_ref[1]
    out_ref[...] = y / (dsum + 1e-16)


def kernel(H_indices, H_values, X, a):
    n_nodes, d = X.shape
    nnz = H_values.shape[0]
    per_tile = nnz // _TILES
    nch = per_tile // _C
    nid_flat = H_indices[0].astype(jnp.int32)
    hid_flat = H_indices[1].astype(jnp.int32)
    nid3 = nid_flat.reshape(_TILES, nch, _C)
    hid3 = hid_flat.reshape(_TILES, nch, _C)
    vals = H_values.astype(jnp.float32)

    def _perm(x):
        return (x.reshape(_TILES, nch, _G, 16).swapaxes(2, 3)
                .reshape(_TILES, nch, _C))

    nidp_flat = _perm(nid3).reshape(-1)
    hid3p = _perm(hid3)
    valsp = _perm(vals.reshape(_TILES, nch, _C)).reshape(-1)

    mesh = plsc.VectorSubcoreMesh(core_axis_name="c", subcore_axis_name="s")
    sc_params = pltpu.CompilerParams(needs_layout_passes=False)

    edge_accum = pl.kernel(
        _edge_accum_body,
        out_type=jax.ShapeDtypeStruct((_NC, NP, D), jnp.float32),
        mesh=mesh,
        compiler_params=sc_params,
        scratch_types=[
            pltpu.VMEM_SHARED((NP, D), jnp.float32),
            pltpu.VMEM((nch, _C), jnp.int32),
            pltpu.VMEM((_C,), jnp.int32),
            pltpu.VMEM((_C,), jnp.int32),
            pltpu.VMEM((_C,), jnp.float32),
            pltpu.VMEM((_C,), jnp.float32),
            pltpu.VMEM((_C, D), jnp.float32),
            pltpu.VMEM((_C, D), jnp.float32),
            pltpu.VMEM((_C, D), jnp.float32),
            pltpu.SemaphoreType.DMA,
            pltpu.SemaphoreType.DMA,
            pltpu.SemaphoreType.DMA,
            pltpu.SemaphoreType.DMA,
            pltpu.SemaphoreType.DMA,
            pltpu.SemaphoreType.DMA,
        ],
    )
    e_parts = edge_accum(nidp_flat, hid3p, valsp, X)

    r_blk = 1024
    et_full, w2 = pl.pallas_call(
        _project_body,
        grid=(NP // r_blk,),
        in_specs=[
            pl.BlockSpec((_NC, r_blk, D), lambda i: (0, i, 0)),
            pl.BlockSpec((1, D), lambda i: (0, 0)),
        ],
        out_specs=[
            pl.BlockSpec((r_blk, D), lambda i: (i, 0)),
            pl.BlockSpec((r_blk, 1), lambda i: (i, 0)),
        ],
        out_shape=[
            jax.ShapeDtypeStruct((NP, D), jnp.float32),
            jax.ShapeDtypeStruct((NP, 1), jnp.float32),
        ],
    )(e_parts, a[d:].reshape(1, d))
    w_tab = w2.reshape(-1)

    attn_accum = pl.kernel(
        _attn_accum_body,
        out_type=[
            jax.ShapeDtypeStruct((_NC, NP, D), jnp.float32),
            jax.ShapeDtypeStruct((_NC, NP), jnp.float32),
        ],
        mesh=mesh,
        compiler_params=sc_params,
        scratch_types=[
            pltpu.VMEM_SHARED((NP, D), jnp.float32),
            pltpu.VMEM_SHARED((NP,), jnp.float32),
            pltpu.VMEM((nch, _C), jnp.int32),
            pltpu.VMEM((per_tile,), jnp.int32),
            pltpu.VMEM((_C,), jnp.float32),
            pltpu.VMEM((_C,), jnp.float32),
            pltpu.VMEM((_C, D), jnp.float32),
            pltpu.VMEM((_C, D), jnp.float32),
            pltpu.VMEM((128,), jnp.float32),
            pltpu.SemaphoreType.DMA,
            pltpu.SemaphoreType.DMA,
            pltpu.SemaphoreType.DMA,
            pltpu.SemaphoreType.DMA,
        ],
    )
    y_parts, d_parts = attn_accum(nid3, hid_flat, w_tab, et_full)

    dp3 = d_parts.reshape(_NC, NP, 1)
    out_pad = pl.pallas_call(
        _finalize_body,
        grid=(NP // r_blk,),
        in_specs=[
            pl.BlockSpec((_NC, r_blk, D), lambda i: (0, i, 0)),
            pl.BlockSpec((_NC, r_blk, 1), lambda i: (0, i, 0)),
        ],
        out_specs=pl.BlockSpec((r_blk, D), lambda i: (i, 0)),
        out_shape=jax.ShapeDtypeStruct((NP, D), jnp.float32),
    )(y_parts, dp3)
    return out_pad[:n_nodes]


# speedup vs baseline: 1.6799x; 1.3889x over previous
import jax
import jax.numpy as jnp
from jax import lax
from jax.experimental import pallas as pl
from jax.experimental.pallas import tpu as pltpu
from jax.experimental.pallas import tpu_sc as plsc

HE = 10000
D = 128
NP = 10240

_NC = 2
_NS = 16
_TILES = _NC * _NS
_C = 80
_G = _C // 16
_NB = 2
_NBA = 3


def _bcast_lane(v16, i):
    return jnp.take_along_axis(v16, jnp.full((16,), i, jnp.int32), axis=0)


def _zero_rows(ref, nrows):
    z = jnp.zeros((16,), jnp.float32)

    def body(r, carry):
        for b in range(D // 16):
            ref[r, pl.ds(b * 16, 16)] = z
        return carry

    lax.fori_loop(0, nrows, body, 0)


def _scale_rows(src, dst, scal16, g):
    for i in range(16):
        bv = _bcast_lane(scal16, i)
        r = g * 16 + i
        for b in range(D // 16):
            dst[r, pl.ds(b * 16, 16)] = src[r, pl.ds(b * 16, 16)] * bv


def _zero_acc_slice(acc, rows_buf, s):
    _zero_rows(rows_buf, _C)
    rpt = NP // _NS
    for j in range(rpt // _C):
        r0 = pl.multiple_of(s * rpt + j * _C, 8)
        pltpu.sync_copy(rows_buf, acc.at[pl.ds(r0, _C)])


def _stage_writeback(acc, parts_hbm, rows_buf, c, s):
    rpt = NP // _NS
    for j in range(rpt // _C):
        r0 = pl.multiple_of(s * rpt + j * _C, 8)
        pltpu.sync_copy(acc.at[pl.ds(r0, _C)], rows_buf)
        pltpu.sync_copy(rows_buf, parts_hbm.at[c, pl.ds(r0, _C)])


def _edge_accum_body(nid_hbm, hid_hbm, val_hbm, x_hbm, eparts_hbm,
                     e_acc, nidb0, nidb1, nidb2, hidb0, hidb1, hidb2,
                     valb0, valb1, valb2, rin0, rin1, rin2, rout,
                     semg0, semg1, semg2, semv0, semv1, semv2,
                     semn0, semn1, semn2, semh0, semh1, semh2):
    c = lax.axis_index("c")
    s = lax.axis_index("s")
    wid = c * _NS + s
    per_tile = nid_hbm.shape[0] // _TILES
    nch = per_tile // _C
    base = wid * per_tile
    rin = (rin0, rin1, rin2)
    nidb = (nidb0, nidb1, nidb2)
    hidb = (hidb0, hidb1, hidb2)
    valb = (valb0, valb1, valb2)
    semg = (semg0, semg1, semg2)
    semv = (semv0, semv1, semv2)
    semn = (semn0, semn1, semn2)
    semh = (semh0, semh1, semh2)

    def start_nid(k, p):
        off = pl.multiple_of(base + k * _C, 8)
        pltpu.async_copy(nid_hbm.at[pl.ds(off, _C)], nidb[p], semn[p])

    def start_val_hid(k, p):
        off = pl.multiple_of(base + k * _C, 8)
        pltpu.async_copy(val_hbm.at[pl.ds(off, _C)], valb[p], semv[p])
        pltpu.async_copy(hid_hbm.at[pl.ds(off, _C)], hidb[p], semh[p])

    def start_gather(p):
        pltpu.make_async_copy(nid_hbm.at[pl.ds(0, _C)], nidb[p], semn[p]).wait()
        pltpu.async_copy(x_hbm.at[nidb[p]], rin[p], semg[p])

    for p in range(_NBA):
        start_nid(p, p)
        start_val_hid(p, p)
    for p in range(_NBA):
        start_gather(p)

    _zero_acc_slice(e_acc, rout, s)
    plsc.subcore_barrier()

    def visit(k, p, prefetch):
        pltpu.make_async_copy(x_hbm.at[nidb[p]], rin[p], semg[p]).wait()
        pltpu.make_async_copy(val_hbm.at[pl.ds(0, _C)], valb[p], semv[p]).wait()
        if prefetch:
            start_nid(k + _NBA, p)
        for g in range(_G):
            v16 = valb[p][pl.ds(g * 16, 16)]
            _scale_rows(rin[p], rout, v16, g)
        pltpu.make_async_copy(hid_hbm.at[pl.ds(0, _C)], hidb[p], semh[p]).wait()
        pltpu.sync_copy(rout, e_acc.at[hidb[p]], add=True)
        if prefetch:
            start_gather(p)
            start_val_hid(k + _NBA, p)

    def steady(kk, carry):
        for b in range(_NBA):
            visit(kk * _NBA + b, b, True)
        return carry

    n_steady = (nch - _NBA) // _NBA
    lax.fori_loop(0, n_steady, steady, 0)
    for k in range(n_steady * _NBA, nch):
        visit(k, k % _NBA, k + _NBA < nch)

    plsc.subcore_barrier()
    _stage_writeback(e_acc, eparts_hbm, rin0, c, s)


def _attn_accum_body(nid_hbm, hid_hbm, w_hbm, et_hbm,
                     yparts_hbm, dparts_hbm,
                     y_acc, d_acc, nid_v, hid_v,
                     wb0, wb1, rin0, rin1, dbuf,
                     semg0, semg1, semw0, semw1):
    c = lax.axis_index("c")
    s = lax.axis_index("s")
    wid = c * _NS + s
    nch = nid_v.shape[0]
    per_tile = nch * _C
    base = wid * per_tile
    rin = (rin0, rin1)
    wb = (wb0, wb1)
    semg = (semg0, semg1)
    semw = (semw0, semw1)

    b0 = pl.multiple_of(base, 8)
    pltpu.sync_copy(nid_hbm.at[wid], nid_v)
    pltpu.sync_copy(hid_hbm.at[pl.ds(b0, per_tile)], hid_v)

    _zero_acc_slice(y_acc, rin0, s)
    z16 = jnp.zeros((16,), jnp.float32)
    rpt = NP // _NS

    def zd(i, carry):
        dbuf[pl.ds(i * 16, 16)] = z16
        return carry

    lax.fori_loop(0, 8, zd, 0)
    d0 = pl.multiple_of(s * rpt, 8)
    for j in range(rpt // 128):
        pltpu.sync_copy(dbuf, d_acc.at[pl.ds(pl.multiple_of(d0 + j * 128, 8), 128)])

    def start_fetch(k, p):
        hslice = hid_v.at[pl.ds(k * _C, _C)]
        pltpu.async_copy(et_hbm.at[hslice], rin[p], semg[p])
        pltpu.async_copy(w_hbm.at[hslice], wb[p], semw[p])

    def wait_fetch(k, p):
        hslice = hid_v.at[pl.ds(k * _C, _C)]
        pltpu.make_async_copy(et_hbm.at[hslice], rin[p], semg[p]).wait()
        pltpu.make_async_copy(w_hbm.at[hslice], wb[p], semw[p]).wait()

    for p in range(_NB):
        start_fetch(p, p)
    plsc.subcore_barrier()

    def visit(k, p, prefetch):
        wait_fetch(k, p)
        pltpu.sync_copy(wb[p], d_acc.at[nid_v.at[k]], add=True)
        pltpu.sync_copy(rin[p], y_acc.at[nid_v.at[k]], add=True)
        if prefetch:
            start_fetch(k + _NB, p)

    def steady(kk, carry):
        visit(kk * _NB, 0, True)
        visit(kk * _NB + 1, 1, True)
        return carry

    lax.fori_loop(0, (nch - 3) // _NB, steady, 0)
    visit(nch - 3, 0, True)
    visit(nch - 2, 1, False)
    visit(nch - 1, 0, False)

    plsc.subcore_barrier()
    _stage_writeback(y_acc, yparts_hbm, rin0, c, s)
    for j in range(rpt // 128):
        dj = pl.multiple_of(d0 + j * 128, 8)
        pltpu.sync_copy(d_acc.at[pl.ds(dj, 128)], dbuf)
        pltpu.sync_copy(dbuf, dparts_hbm.at[c, pl.ds(dj, 128)])


def _project_body(ep_ref, a2_ref, et_ref, w_ref):
    ep = ep_ref[0] + ep_ref[1]
    t = jnp.sum(ep * a2_ref[...], axis=1, keepdims=True)
    w = jnp.exp(t)
    w_ref[...] = w
    et_ref[...] = ep * w


def _finalize_body(yp_ref, dp_ref, out_ref):
    y = yp_ref[0] + yp_ref[1]
    dsum = dp_ref[0] + dp_ref[1]
    out_ref[...] = y / (dsum + 1e-16)


def kernel(H_indices, H_values, X, a):
    n_nodes, d = X.shape
    nnz = H_values.shape[0]
    per_tile = nnz // _TILES
    nch = per_tile // _C
    nid_flat = H_indices[0].astype(jnp.int32)
    hid_flat = H_indices[1].astype(jnp.int32)
    nid3 = nid_flat.reshape(_TILES, nch, _C)
    hid3 = hid_flat.reshape(_TILES, nch, _C)
    vals = H_values.astype(jnp.float32)


    mesh = plsc.VectorSubcoreMesh(core_axis_name="c", subcore_axis_name="s")
    sc_params = pltpu.CompilerParams(needs_layout_passes=False)

    edge_accum = pl.kernel(
        _edge_accum_body,
        out_type=jax.ShapeDtypeStruct((_NC, NP, D), jnp.float32),
        mesh=mesh,
        compiler_params=sc_params,
        scratch_types=(
            [pltpu.VMEM_SHARED((NP, D), jnp.float32)]
            + [pltpu.VMEM((_C,), jnp.int32) for _ in range(2 * _NBA)]
            + [pltpu.VMEM((_C,), jnp.float32) for _ in range(_NBA)]
            + [pltpu.VMEM((_C, D), jnp.float32) for _ in range(_NBA + 1)]
            + [pltpu.SemaphoreType.DMA for _ in range(4 * _NBA)]
        ),
    )
    e_parts = edge_accum(nid_flat, hid_flat, vals, X)

    r_blk = 1024
    et_full, w2 = pl.pallas_call(
        _project_body,
        grid=(NP // r_blk,),
        in_specs=[
            pl.BlockSpec((_NC, r_blk, D), lambda i: (0, i, 0)),
            pl.BlockSpec((1, D), lambda i: (0, 0)),
        ],
        out_specs=[
            pl.BlockSpec((r_blk, D), lambda i: (i, 0)),
            pl.BlockSpec((r_blk, 1), lambda i: (i, 0)),
        ],
        out_shape=[
            jax.ShapeDtypeStruct((NP, D), jnp.float32),
            jax.ShapeDtypeStruct((NP, 1), jnp.float32),
        ],
    )(e_parts, a[d:].reshape(1, d))
    w_tab = w2.reshape(-1)

    attn_accum = pl.kernel(
        _attn_accum_body,
        out_type=[
            jax.ShapeDtypeStruct((_NC, NP, D), jnp.float32),
            jax.ShapeDtypeStruct((_NC, NP), jnp.float32),
        ],
        mesh=mesh,
        compiler_params=sc_params,
        scratch_types=[
            pltpu.VMEM_SHARED((NP, D), jnp.float32),
            pltpu.VMEM_SHARED((NP,), jnp.float32),
            pltpu.VMEM((nch, _C), jnp.int32),
            pltpu.VMEM((per_tile,), jnp.int32),
            pltpu.VMEM((_C,), jnp.float32),
            pltpu.VMEM((_C,), jnp.float32),
            pltpu.VMEM((_C, D), jnp.float32),
            pltpu.VMEM((_C, D), jnp.float32),
            pltpu.VMEM((128,), jnp.float32),
            pltpu.SemaphoreType.DMA,
            pltpu.SemaphoreType.DMA,
            pltpu.SemaphoreType.DMA,
            pltpu.SemaphoreType.DMA,
        ],
    )
    y_parts, d_parts = attn_accum(nid3, hid_flat, w_tab, et_full)

    dp3 = d_parts.reshape(_NC, NP, 1)
    out_pad = pl.pallas_call(
        _finalize_body,
        grid=(NP // r_blk,),
        in_specs=[
            pl.BlockSpec((_NC, r_blk, D), lambda i: (0, i, 0)),
            pl.BlockSpec((_NC, r_blk, 1), lambda i: (0, i, 0)),
        ],
        out_specs=pl.BlockSpec((r_blk, D), lambda i: (i, 0)),
        out_shape=jax.ShapeDtypeStruct((NP, D), jnp.float32),
    )(y_parts, dp3)
    return out_pad[:n_nodes]
